# baseline (speedup 1.0000x reference)
import functools

import jax
import jax.numpy as jnp
from jax import lax
from jax.experimental import pallas as pl
from jax.experimental.pallas import tpu as pltpu
from jax.experimental.pallas import tpu_sc as plsc

N = 10000
E = 160000
D_IN = 256
D_HID = 256
D_OUT = 64

NP = 10240
BM = 1024
NC = 2
NS = 16
NPT = NP // NS

_MESH = plsc.VectorSubcoreMesh(core_axis_name="c", subcore_axis_name="s")


_EPW = E // (NC * NS)


def _deg_body(src_hbm, dst_hbm, degp_hbm, idx_s, idx_d, acc_s, acc_d,
              shr, red, pbuf, sem):
    c = lax.axis_index("c")
    s = lax.axis_index("s")
    w = c * NS + s

    zero16 = jnp.zeros((16,), jnp.float32)

    def zero_body(j, _):
        acc_s[pl.ds(j * 16, 16)] = zero16
        acc_d[pl.ds(j * 16, 16)] = zero16
        return _
    lax.fori_loop(0, NP // 16, zero_body, None)

    pltpu.sync_copy(src_hbm.at[pl.ds(w * _EPW, _EPW)], idx_s)
    pltpu.sync_copy(dst_hbm.at[pl.ds(w * _EPW, _EPW)], idx_d)

    ones16 = jnp.ones((16,), jnp.float32)
    nfull = _EPW // 16

    def scat_body(j, _):
        i_s = idx_s[pl.ds(j * 16, 16)]
        i_d = idx_d[pl.ds(j * 16, 16)]
        plsc.addupdate_scatter(acc_s, [i_s], ones16)
        plsc.addupdate_scatter(acc_d, [i_d], ones16)
        return _
    lax.fori_loop(0, nfull, scat_body, None)

    rem = _EPW - nfull * 16
    if rem:
        tailmask = lax.iota(jnp.int32, 16) >= (16 - rem)
        i_s = idx_s[pl.ds(_EPW - 16, 16)]
        i_d = idx_d[pl.ds(_EPW - 16, 16)]
        plsc.addupdate_scatter(acc_s, [i_s], ones16, mask=tailmask)
        plsc.addupdate_scatter(acc_d, [i_d], ones16, mask=tailmask)

    pltpu.sync_copy(acc_s, shr.at[0, s])
    pltpu.sync_copy(acc_d, shr.at[1, s])
    plsc.subcore_barrier()

    for a in range(2):
        pltpu.sync_copy(shr.at[a, :, pl.ds(s * NPT, NPT)], red)

        def red_body(q, _):
            v = red[0, pl.ds(q * 16, 16)]
            for r in range(1, NS):
                v = v + red[r, pl.ds(q * 16, 16)]
            pbuf[pl.ds(q * 16, 16)] = v
            return _
        lax.fori_loop(0, NPT // 16, red_body, None)
        pltpu.sync_copy(pbuf, degp_hbm.at[a, c, pl.ds(s * NPT, NPT)])


@functools.partial(
    pl.kernel,
    out_type=jax.ShapeDtypeStruct((2, NC, NP), jnp.float32),
    mesh=_MESH,
    compiler_params=pltpu.CompilerParams(needs_layout_passes=False),
    scratch_types=[
        pltpu.VMEM((_EPW,), jnp.int32),
        pltpu.VMEM((_EPW,), jnp.int32),
        pltpu.VMEM((NP,), jnp.float32),
        pltpu.VMEM((NP,), jnp.float32),
        pltpu.VMEM_SHARED((2, NS, NP), jnp.float32),
        pltpu.VMEM((NS, NPT), jnp.float32),
        pltpu.VMEM((NPT,), jnp.float32),
        pltpu.SemaphoreType.DMA,
    ],
)
def _deg_kernel(src_hbm, dst_hbm, degp_hbm, idx_s, idx_d, acc_s, acc_d,
                shr, red, pbuf, sem):
    _deg_body(src_hbm, dst_hbm, degp_hbm, idx_s, idx_d, acc_s, acc_d,
              shr, red, pbuf, sem)


_EPT = E // NS
_CK = 80
_NCH = _EPT // _CK


def _zero_rows(rows, nrow):
    zero16 = jnp.zeros((16,), jnp.float32)

    def zrow(r, _):
        def zcol(q, __):
            rows[r, pl.ds(q * 16, 16)] = zero16
            return __
        return lax.fori_loop(0, 128 // 16, zcol, _)
    lax.fori_loop(0, nrow, zrow, None)


def _zero_acc_slice(rows, acc, s, ck):
    nfull = NPT // ck
    rem = NPT - nfull * ck

    def zacc(j, _):
        pltpu.sync_copy(rows, acc.at[pl.ds(s * NPT + j * ck, ck)])
        return _
    lax.fori_loop(0, nfull, zacc, None)
    if rem:
        pltpu.sync_copy(rows.at[pl.ds(0, rem)],
                        acc.at[pl.ds(s * NPT + nfull * ck, rem)])


_NB = 4
_NI = 8


def _prop_pipeline_deep(h_hbm, acc, sd_idx, ibs, rbs, isems, gsems, ssems,
                        nch):
    def fire_i(i, q):
        pltpu.async_copy(sd_idx.at[i], ibs[q % _NI], isems[q % _NI])

    def wait_i(i, q):
        pltpu.make_async_copy(sd_idx.at[i], ibs[q % _NI],
                              isems[q % _NI]).wait()

    def fire_g(q):
        pltpu.async_copy(h_hbm.at[ibs[q % _NI].at[0]], rbs[q % _NB],
                         gsems[q % _NB])

    def wait_g(q):
        pltpu.make_async_copy(h_hbm.at[ibs[q % _NI].at[0]], rbs[q % _NB],
                              gsems[q % _NB]).wait()

    def fire_s(q):
        pltpu.async_copy(rbs[q % _NB], acc.at[ibs[q % _NI].at[1]],
                         ssems[q % _NB], add=True)

    def wait_s(q):
        pltpu.make_async_copy(rbs[q % _NB], acc.at[ibs[q % _NI].at[1]],
                              ssems[q % _NB]).wait()

    for j in range(min(6, nch)):
        fire_i(j, j)
    for j in range(min(2, nch)):
        wait_i(j, j)
        fire_g(j)

    def step(i, q, static):
        wait_g(q)
        fire_s(q)

        def lazy_waits():
            wait_s(q - 2)

        def ahead_g():
            wait_i(i + 2, q + 2)
            fire_g(q + 2)

        def ahead_i():
            fire_i(i + 6, q + 6)

        if static:
            if i >= 2:
                lazy_waits()
            if i + 2 < nch:
                ahead_g()
            if i + 6 < nch:
                ahead_i()
        else:
            pl.when(i >= 2)(lazy_waits)
            pl.when(i + 2 < nch)(ahead_g)
            pl.when(i + 6 < nch)(ahead_i)

    nloop = nch // _NI

    def body(p, _):
        base = p * _NI
        for q in range(_NI):
            step(base + q, q, False)
        return _
    lax.fori_loop(0, nloop, body, None)

    for i in range(nloop * _NI, nch):
        step(i, i % _NI, True)

    for i in range(max(nch - 2, 0), nch):
        wait_s(i % _NI)


def _prop_scratch(ck):
    return (
        [pltpu.VMEM((2, ck), jnp.int32) for _ in range(_NI)]
        + [pltpu.VMEM((ck, 128), jnp.float32) for _ in range(_NB)]
        + [pltpu.VMEM_SHARED((NP, 128), jnp.float32)]
        + [pltpu.SemaphoreType.DMA for _ in range(_NI + 2 * _NB)]
    )


def _prop_body(h_hbm, sd_hbm, agg_hbm, out_at, ck, nch, *scr):
    c = lax.axis_index("c")
    s = lax.axis_index("s")

    ibs = scr[:_NI]
    rbs = scr[_NI:_NI + _NB]
    acc = scr[_NI + _NB]
    isems = scr[_NI + _NB + 1:2 * _NI + _NB + 1]
    gsems = scr[2 * _NI + _NB + 1:2 * _NI + 2 * _NB + 1]
    ssems = scr[2 * _NI + 2 * _NB + 1:]

    _zero_rows(rbs[0], ck)
    _zero_acc_slice(rbs[0], acc, s, ck)
    plsc.subcore_barrier()

    _prop_pipeline_deep(h_hbm, acc, sd_hbm.at[c, s], ibs, rbs,
                        isems, gsems, ssems, nch)

    plsc.subcore_barrier()
    pltpu.sync_copy(acc.at[pl.ds(s * NPT, NPT)], out_at(agg_hbm, c, s))


@functools.partial(
    pl.kernel,
    out_type=jax.ShapeDtypeStruct((2 * NP, 128), jnp.float32),
    mesh=_MESH,
    scratch_types=_prop_scratch(_CK),
)
def _prop_col_kernel(h_hbm, sd_hbm, agg_hbm, *scr):
    _prop_body(h_hbm, sd_hbm, agg_hbm,
               lambda ref, c, s: ref.at[pl.ds(c * NP + s * NPT, NPT)],
               _CK, _NCH, *scr)


_EPW3 = E // (NC * NS)
_CK3 = 50
_NCH3 = _EPW3 // _CK3


@functools.partial(
    pl.kernel,
    out_type=jax.ShapeDtypeStruct((NC, NP, 128), jnp.float32),
    mesh=_MESH,
    scratch_types=_prop_scratch(_CK3),
)
def _prop_edge_kernel(h_hbm, sd_hbm, aggp_hbm, *scr):
    _prop_body(h_hbm, sd_hbm, aggp_hbm,
               lambda ref, c, s: ref.at[c, pl.ds(s * NPT, NPT)],
               _CK3, _NCH3, *scr)


def _mm1_body(x_ref, dsa_ref, dsb_ref, w_ref, o_ref):
    ns = lax.rsqrt(jnp.maximum(dsa_ref[...] + dsb_ref[...], 1.0))
    o_ref[...] = jnp.dot(x_ref[...] * ns, w_ref[...],
                         preferred_element_type=jnp.float32)


def _mm1(x, dsa, dsb, w1):
    nb = NP // BM
    return pl.pallas_call(
        _mm1_body,
        grid=(nb, 2),
        in_specs=[
            pl.BlockSpec((BM, D_IN), lambda i, j: (i, 0)),
            pl.BlockSpec((BM, 1), lambda i, j: (i, 0)),
            pl.BlockSpec((BM, 1), lambda i, j: (i, 0)),
            pl.BlockSpec((D_IN, 128), lambda i, j: (0, j)),
        ],
        out_specs=pl.BlockSpec((BM, 128), lambda i, j: (j * nb + i, 0)),
        out_shape=jax.ShapeDtypeStruct((2 * NP, 128), jnp.float32),
    )(x, dsa, dsb, w1)


def _mm_mid_body(a_ref, dsa_ref, dsb_ref, dda_ref, ddb_ref, b_ref, w_ref,
                 o_ref):
    k = pl.program_id(2)
    ns = lax.rsqrt(jnp.maximum(dsa_ref[...] + dsb_ref[...], 1.0))
    nd = lax.rsqrt(jnp.maximum(dda_ref[...] + ddb_ref[...], 1.0))
    xb = jnp.maximum(a_ref[...] * (ns * nd) + ns * b_ref[0], 0.0)
    part = jnp.dot(xb, w_ref[...], preferred_element_type=jnp.float32)

    @pl.when(k == 0)
    def _():
        o_ref[...] = part

    @pl.when(k != 0)
    def _():
        o_ref[...] += part


def _mm_mid(agg, dsa, dsb, dda, ddb, b2d, w):
    nb = NP // BM
    return pl.pallas_call(
        _mm_mid_body,
        grid=(nb, 2, 2),
        in_specs=[
            pl.BlockSpec((BM, 128), lambda i, j, k: (k * nb + i, 0)),
            pl.BlockSpec((BM, 1), lambda i, j, k: (i, 0)),
            pl.BlockSpec((BM, 1), lambda i, j, k: (i, 0)),
            pl.BlockSpec((BM, 1), lambda i, j, k: (i, 0)),
            pl.BlockSpec((BM, 1), lambda i, j, k: (i, 0)),
            pl.BlockSpec((1, 1, 128), lambda i, j, k: (k, 0, 0)),
            pl.BlockSpec((128, 128), lambda i, j, k: (k, j)),
        ],
        out_specs=pl.BlockSpec((BM, 128), lambda i, j, k: (j * nb + i, 0)),
        out_shape=jax.ShapeDtypeStruct((2 * NP, 128), jnp.float32),
    )(agg, dsa, dsb, dda, ddb, b2d, w)


def _mm3_body(a_ref, dsa_ref, dsb_ref, dda_ref, ddb_ref, b_ref, w_ref,
              o_ref):
    k = pl.program_id(1)
    ns = lax.rsqrt(jnp.maximum(dsa_ref[...] + dsb_ref[...], 1.0))
    nd = lax.rsqrt(jnp.maximum(dda_ref[...] + ddb_ref[...], 1.0))
    xb = jnp.maximum(a_ref[...] * (ns * nd) + ns * b_ref[0], 0.0)
    part = jnp.dot(xb, w_ref[...], preferred_element_type=jnp.float32)

    @pl.when(k == 0)
    def _():
        o_ref[...] = part

    @pl.when(k != 0)
    def _():
        o_ref[...] += part


def _mm3(agg, dsa, dsb, dda, ddb, b2d, w3):
    nb = NP // BM
    return pl.pallas_call(
        _mm3_body,
        grid=(nb, 2),
        in_specs=[
            pl.BlockSpec((BM, 128), lambda i, k: (k * nb + i, 0)),
            pl.BlockSpec((BM, 1), lambda i, k: (i, 0)),
            pl.BlockSpec((BM, 1), lambda i, k: (i, 0)),
            pl.BlockSpec((BM, 1), lambda i, k: (i, 0)),
            pl.BlockSpec((BM, 1), lambda i, k: (i, 0)),
            pl.BlockSpec((1, 1, 128), lambda i, k: (k, 0, 0)),
            pl.BlockSpec((128, 128), lambda i, k: (k, 0)),
        ],
        out_specs=pl.BlockSpec((BM, 128), lambda i, k: (i, 0)),
        out_shape=jax.ShapeDtypeStruct((NP, 128), jnp.float32),
    )(agg, dsa, dsb, dda, ddb, b2d, w3)


def _epi_body(ap_ref, dda_ref, ddb_ref, b_ref, o_ref):
    nd = lax.rsqrt(jnp.maximum(dda_ref[...] + ddb_ref[...], 1.0))
    o_ref[...] = (ap_ref[0, :, :D_OUT] + ap_ref[1, :, :D_OUT]) * nd + b_ref[...]


def _epi(aggp, dda, ddb, b3):
    nb = NP // BM
    return pl.pallas_call(
        _epi_body,
        grid=(nb,),
        in_specs=[
            pl.BlockSpec((2, BM, 128), lambda i: (0, i, 0)),
            pl.BlockSpec((BM, 1), lambda i: (i, 0)),
            pl.BlockSpec((BM, 1), lambda i: (i, 0)),
            pl.BlockSpec((1, D_OUT), lambda i: (0, 0)),
        ],
        out_specs=pl.BlockSpec((BM, D_OUT), lambda i: (i, 0)),
        out_shape=jax.ShapeDtypeStruct((NP, D_OUT), jnp.float32),
    )(aggp, dda, ddb, b3)


def kernel(features, edge_index, W1, b1, W2, b2, W3, b3):
    src = edge_index[0]
    dst = edge_index[1]
    srcoff = jnp.concatenate([src, src + NP]).reshape(NC, NS, _NCH, _CK)
    dstb = jnp.broadcast_to(dst.reshape(1, NS, _NCH, _CK),
                            (NC, NS, _NCH, _CK))
    sd_col = jnp.stack([srcoff, dstb], axis=3)
    sd_edge = jnp.stack([src.reshape(NC, NS, _NCH3, _CK3),
                         dst.reshape(NC, NS, _NCH3, _CK3)], axis=3)

    x = jnp.pad(features, ((0, NP - N), (0, 0)))

    degp = _deg_kernel(src, dst)
    dsa = degp[0, 0].reshape(NP, 1)
    dsb = degp[0, 1].reshape(NP, 1)
    dda = degp[1, 0].reshape(NP, 1)
    ddb = degp[1, 1].reshape(NP, 1)

    h1 = _mm1(x, dsa, dsb, W1)
    agg1 = _prop_col_kernel(h1, sd_col)
    h2 = _mm_mid(agg1, dsa, dsb, dda, ddb, b1.reshape(2, 1, 128), W2)
    agg2 = _prop_col_kernel(h2, sd_col)
    w3p = jnp.pad(W3, ((0, 0), (0, 128 - D_OUT)))
    h3 = _mm3(agg2, dsa, dsb, dda, ddb, b2.reshape(2, 1, 128), w3p)
    aggp3 = _prop_edge_kernel(h3, sd_edge)
    out = _epi(aggp3, dda, ddb, b3.reshape(1, D_OUT))
    return out[:N]


# speedup vs baseline: 1.0656x; 1.0656x over previous
import functools

import jax
import jax.numpy as jnp
from jax import lax
from jax.experimental import pallas as pl
from jax.experimental.pallas import tpu as pltpu
from jax.experimental.pallas import tpu_sc as plsc

N = 10000
E = 160000
D_IN = 256
D_HID = 256
D_OUT = 64

NP = 10240
BM = 1024
NC = 2
NS = 16
NPT = NP // NS

_MESH = plsc.VectorSubcoreMesh(core_axis_name="c", subcore_axis_name="s")


_EPW = E // (NC * NS)


def _deg_body(src_hbm, dst_hbm, degp_hbm, idx_s, idx_d, acc_s, acc_d,
              shr, red, pbuf, sem):
    c = lax.axis_index("c")
    s = lax.axis_index("s")
    w = c * NS + s

    zero16 = jnp.zeros((16,), jnp.float32)

    def zero_body(j, _):
        acc_s[pl.ds(j * 16, 16)] = zero16
        acc_d[pl.ds(j * 16, 16)] = zero16
        return _
    lax.fori_loop(0, NP // 16, zero_body, None)

    pltpu.sync_copy(src_hbm.at[pl.ds(w * _EPW, _EPW)], idx_s)
    pltpu.sync_copy(dst_hbm.at[pl.ds(w * _EPW, _EPW)], idx_d)

    ones16 = jnp.ones((16,), jnp.float32)
    nfull = _EPW // 16

    def scat_body(j, _):
        i_s = idx_s[pl.ds(j * 16, 16)]
        i_d = idx_d[pl.ds(j * 16, 16)]
        plsc.addupdate_scatter(acc_s, [i_s], ones16)
        plsc.addupdate_scatter(acc_d, [i_d], ones16)
        return _
    lax.fori_loop(0, nfull, scat_body, None)

    rem = _EPW - nfull * 16
    if rem:
        tailmask = lax.iota(jnp.int32, 16) >= (16 - rem)
        i_s = idx_s[pl.ds(_EPW - 16, 16)]
        i_d = idx_d[pl.ds(_EPW - 16, 16)]
        plsc.addupdate_scatter(acc_s, [i_s], ones16, mask=tailmask)
        plsc.addupdate_scatter(acc_d, [i_d], ones16, mask=tailmask)

    pltpu.sync_copy(acc_s, shr.at[0, s])
    pltpu.sync_copy(acc_d, shr.at[1, s])
    plsc.subcore_barrier()

    for a in range(2):
        pltpu.sync_copy(shr.at[a, :, pl.ds(s * NPT, NPT)], red)

        def red_body(q, _):
            v = red[0, pl.ds(q * 16, 16)]
            for r in range(1, NS):
                v = v + red[r, pl.ds(q * 16, 16)]
            pbuf[pl.ds(q * 16, 16)] = v
            return _
        lax.fori_loop(0, NPT // 16, red_body, None)
        pltpu.sync_copy(pbuf, degp_hbm.at[a, c, pl.ds(s * NPT, NPT)])


@functools.partial(
    pl.kernel,
    out_type=jax.ShapeDtypeStruct((2, NC, NP), jnp.float32),
    mesh=_MESH,
    compiler_params=pltpu.CompilerParams(needs_layout_passes=False),
    scratch_types=[
        pltpu.VMEM((_EPW,), jnp.int32),
        pltpu.VMEM((_EPW,), jnp.int32),
        pltpu.VMEM((NP,), jnp.float32),
        pltpu.VMEM((NP,), jnp.float32),
        pltpu.VMEM_SHARED((2, NS, NP), jnp.float32),
        pltpu.VMEM((NS, NPT), jnp.float32),
        pltpu.VMEM((NPT,), jnp.float32),
        pltpu.SemaphoreType.DMA,
    ],
)
def _deg_kernel(src_hbm, dst_hbm, degp_hbm, idx_s, idx_d, acc_s, acc_d,
                shr, red, pbuf, sem):
    _deg_body(src_hbm, dst_hbm, degp_hbm, idx_s, idx_d, acc_s, acc_d,
              shr, red, pbuf, sem)


_EPT = E // NS
_CK = 125
_NCH = _EPT // _CK


def _zero_rows(rows, nrow):
    zero16 = jnp.zeros((16,), jnp.float32)

    def zrow(r, _):
        def zcol(q, __):
            rows[r, pl.ds(q * 16, 16)] = zero16
            return __
        return lax.fori_loop(0, 128 // 16, zcol, _)
    lax.fori_loop(0, nrow, zrow, None)


def _zero_acc_slice(rows, acc, s, ck):
    nfull = NPT // ck
    rem = NPT - nfull * ck

    def zacc(j, _):
        pltpu.sync_copy(rows, acc.at[pl.ds(s * NPT + j * ck, ck)])
        return _
    lax.fori_loop(0, nfull, zacc, None)
    if rem:
        pltpu.sync_copy(rows.at[pl.ds(0, rem)],
                        acc.at[pl.ds(s * NPT + nfull * ck, rem)])


def _prop_pipeline_streamed(h_hbm, acc, src_idx, idx_d2, is0, is1, r0, r1,
                            gs0, gs1, ss0, ss1, iss0, iss1, nch):
    def fire_i(i, ib, isem):
        pltpu.async_copy(src_idx.at[i], ib, isem)

    def wait_i(i, ib, isem):
        pltpu.make_async_copy(src_idx.at[i], ib, isem).wait()

    def fire_g(ib, buf, gsem):
        pltpu.async_copy(h_hbm.at[ib], buf, gsem)

    def wait_g(ib, buf, gsem):
        pltpu.make_async_copy(h_hbm.at[ib], buf, gsem).wait()

    def fire_s(i, buf, ssem):
        pltpu.async_copy(buf, acc.at[idx_d2.at[i]], ssem, add=True)

    def wait_s(i, buf, ssem):
        pltpu.make_async_copy(buf, acc.at[idx_d2.at[i]], ssem).wait()

    fire_i(0, is0, iss0)
    fire_i(1, is1, iss1)
    wait_i(0, is0, iss0)
    fire_g(is0, r0, gs0)
    wait_i(1, is1, iss1)
    fire_g(is1, r1, gs1)

    def half(i, ib, isem, rb, gsem, ssem):
        wait_g(ib, rb, gsem)

        @pl.when(i + 2 < nch)
        def _():
            fire_i(i + 2, ib, isem)

        fire_s(i, rb, ssem)
        wait_s(i, rb, ssem)

        @pl.when(i + 2 < nch)
        def _():
            wait_i(i + 2, ib, isem)
            fire_g(ib, rb, gsem)

    def body(p, _):
        half(2 * p, is0, iss0, r0, gs0, ss0)
        half(2 * p + 1, is1, iss1, r1, gs1, ss1)
        return _
    lax.fori_loop(0, nch // 2, body, None)


def _prop_scratch(ck, nch):
    return [
        pltpu.VMEM((nch, ck), jnp.int32),
        pltpu.VMEM((ck,), jnp.int32),
        pltpu.VMEM((ck,), jnp.int32),
        pltpu.VMEM((ck, 128), jnp.float32),
        pltpu.VMEM((ck, 128), jnp.float32),
        pltpu.VMEM_SHARED((NP, 128), jnp.float32),
        pltpu.SemaphoreType.DMA,
        pltpu.SemaphoreType.DMA,
        pltpu.SemaphoreType.DMA,
        pltpu.SemaphoreType.DMA,
        pltpu.SemaphoreType.DMA,
        pltpu.SemaphoreType.DMA,
    ]


def _prop_body(h_hbm, src4_hbm, dst3_at, agg_out_at, ck, nch,
               idx_d2, is0, is1, r0, r1, acc, gs0, gs1, ss0, ss1, iss0,
               iss1):
    c = lax.axis_index("c")
    s = lax.axis_index("s")

    pltpu.sync_copy(dst3_at(c, s), idx_d2)

    _zero_rows(r0, ck)
    _zero_acc_slice(r0, acc, s, ck)
    plsc.subcore_barrier()

    _prop_pipeline_streamed(h_hbm, acc, src4_hbm.at[c, s], idx_d2,
                            is0, is1, r0, r1, gs0, gs1, ss0, ss1, iss0, iss1,
                            nch)

    plsc.subcore_barrier()
    pltpu.sync_copy(acc.at[pl.ds(s * NPT, NPT)], agg_out_at(c, s))


@functools.partial(
    pl.kernel,
    out_type=jax.ShapeDtypeStruct((2 * NP, 128), jnp.float32),
    mesh=_MESH,
    scratch_types=_prop_scratch(_CK, _NCH),
)
def _prop_col_kernel(h_hbm, src4_hbm, dst3_hbm, agg_hbm, *scr):
    _prop_body(h_hbm, src4_hbm,
               lambda c, s: dst3_hbm.at[s],
               lambda c, s: agg_hbm.at[pl.ds(c * NP + s * NPT, NPT)],
               _CK, _NCH, *scr)


_EPW3 = E // (NC * NS)
_CK3 = 125
_NCH3 = _EPW3 // _CK3


@functools.partial(
    pl.kernel,
    out_type=jax.ShapeDtypeStruct((NC, NP, 128), jnp.float32),
    mesh=_MESH,
    scratch_types=_prop_scratch(_CK3, _NCH3),
)
def _prop_edge_kernel(h_hbm, src4_hbm, dst4_hbm, aggp_hbm, *scr):
    _prop_body(h_hbm, src4_hbm,
               lambda c, s: dst4_hbm.at[c, s],
               lambda c, s: aggp_hbm.at[c, pl.ds(s * NPT, NPT)],
               _CK3, _NCH3, *scr)


def _mm1_body(x_ref, dsa_ref, dsb_ref, w_ref, o_ref):
    ns = lax.rsqrt(jnp.maximum(dsa_ref[...] + dsb_ref[...], 1.0))
    o_ref[...] = jnp.dot(x_ref[...] * ns, w_ref[...],
                         preferred_element_type=jnp.float32)


def _mm1(x, dsa, dsb, w1):
    nb = NP // BM
    return pl.pallas_call(
        _mm1_body,
        grid=(nb, 2),
        in_specs=[
            pl.BlockSpec((BM, D_IN), lambda i, j: (i, 0)),
            pl.BlockSpec((BM, 1), lambda i, j: (i, 0)),
            pl.BlockSpec((BM, 1), lambda i, j: (i, 0)),
            pl.BlockSpec((D_IN, 128), lambda i, j: (0, j)),
        ],
        out_specs=pl.BlockSpec((BM, 128), lambda i, j: (j * nb + i, 0)),
        out_shape=jax.ShapeDtypeStruct((2 * NP, 128), jnp.float32),
    )(x, dsa, dsb, w1)


def _mm_mid_body(a_ref, dsa_ref, dsb_ref, dda_ref, ddb_ref, b_ref, w_ref,
                 o_ref):
    k = pl.program_id(2)
    ns = lax.rsqrt(jnp.maximum(dsa_ref[...] + dsb_ref[...], 1.0))
    nd = lax.rsqrt(jnp.maximum(dda_ref[...] + ddb_ref[...], 1.0))
    xb = jnp.maximum(a_ref[...] * (ns * nd) + ns * b_ref[0], 0.0)
    part = jnp.dot(xb, w_ref[...], preferred_element_type=jnp.float32)

    @pl.when(k == 0)
    def _():
        o_ref[...] = part

    @pl.when(k != 0)
    def _():
        o_ref[...] += part


def _mm_mid(agg, dsa, dsb, dda, ddb, b2d, w):
    nb = NP // BM
    return pl.pallas_call(
        _mm_mid_body,
        grid=(nb, 2, 2),
        in_specs=[
            pl.BlockSpec((BM, 128), lambda i, j, k: (k * nb + i, 0)),
            pl.BlockSpec((BM, 1), lambda i, j, k: (i, 0)),
            pl.BlockSpec((BM, 1), lambda i, j, k: (i, 0)),
            pl.BlockSpec((BM, 1), lambda i, j, k: (i, 0)),
            pl.BlockSpec((BM, 1), lambda i, j, k: (i, 0)),
            pl.BlockSpec((1, 1, 128), lambda i, j, k: (k, 0, 0)),
            pl.BlockSpec((128, 128), lambda i, j, k: (k, j)),
        ],
        out_specs=pl.BlockSpec((BM, 128), lambda i, j, k: (j * nb + i, 0)),
        out_shape=jax.ShapeDtypeStruct((2 * NP, 128), jnp.float32),
    )(agg, dsa, dsb, dda, ddb, b2d, w)


def _mm3_body(a_ref, dsa_ref, dsb_ref, dda_ref, ddb_ref, b_ref, w_ref,
              o_ref):
    k = pl.program_id(1)
    ns = lax.rsqrt(jnp.maximum(dsa_ref[...] + dsb_ref[...], 1.0))
    nd = lax.rsqrt(jnp.maximum(dda_ref[...] + ddb_ref[...], 1.0))
    xb = jnp.maximum(a_ref[...] * (ns * nd) + ns * b_ref[0], 0.0)
    part = jnp.dot(xb, w_ref[...], preferred_element_type=jnp.float32)

    @pl.when(k == 0)
    def _():
        o_ref[...] = part

    @pl.when(k != 0)
    def _():
        o_ref[...] += part


def _mm3(agg, dsa, dsb, dda, ddb, b2d, w3):
    nb = NP // BM
    return pl.pallas_call(
        _mm3_body,
        grid=(nb, 2),
        in_specs=[
            pl.BlockSpec((BM, 128), lambda i, k: (k * nb + i, 0)),
            pl.BlockSpec((BM, 1), lambda i, k: (i, 0)),
            pl.BlockSpec((BM, 1), lambda i, k: (i, 0)),
            pl.BlockSpec((BM, 1), lambda i, k: (i, 0)),
            pl.BlockSpec((BM, 1), lambda i, k: (i, 0)),
            pl.BlockSpec((1, 1, 128), lambda i, k: (k, 0, 0)),
            pl.BlockSpec((128, 128), lambda i, k: (k, 0)),
        ],
        out_specs=pl.BlockSpec((BM, 128), lambda i, k: (i, 0)),
        out_shape=jax.ShapeDtypeStruct((NP, 128), jnp.float32),
    )(agg, dsa, dsb, dda, ddb, b2d, w3)


def _epi_body(ap_ref, dda_ref, ddb_ref, b_ref, o_ref):
    nd = lax.rsqrt(jnp.maximum(dda_ref[...] + ddb_ref[...], 1.0))
    o_ref[...] = (ap_ref[0, :, :D_OUT] + ap_ref[1, :, :D_OUT]) * nd + b_ref[...]


def _epi(aggp, dda, ddb, b3):
    nb = NP // BM
    return pl.pallas_call(
        _epi_body,
        grid=(nb,),
        in_specs=[
            pl.BlockSpec((2, BM, 128), lambda i: (0, i, 0)),
            pl.BlockSpec((BM, 1), lambda i: (i, 0)),
            pl.BlockSpec((BM, 1), lambda i: (i, 0)),
            pl.BlockSpec((1, D_OUT), lambda i: (0, 0)),
        ],
        out_specs=pl.BlockSpec((BM, D_OUT), lambda i: (i, 0)),
        out_shape=jax.ShapeDtypeStruct((NP, D_OUT), jnp.float32),
    )(aggp, dda, ddb, b3)


def kernel(features, edge_index, W1, b1, W2, b2, W3, b3):
    src = edge_index[0]
    dst = edge_index[1]
    srcoff = jnp.concatenate([src, src + NP]).reshape(NC, NS, _NCH, _CK)
    dst3 = dst.reshape(NS, _NCH, _CK)
    src4 = src.reshape(NC, NS, _NCH3, _CK3)
    dst4 = dst.reshape(NC, NS, _NCH3, _CK3)

    x = jnp.pad(features, ((0, NP - N), (0, 0)))

    degp = _deg_kernel(src, dst)
    dsa = degp[0, 0].reshape(NP, 1)
    dsb = degp[0, 1].reshape(NP, 1)
    dda = degp[1, 0].reshape(NP, 1)
    ddb = degp[1, 1].reshape(NP, 1)

    h1 = _mm1(x, dsa, dsb, W1)
    agg1 = _prop_col_kernel(h1, srcoff, dst3)
    h2 = _mm_mid(agg1, dsa, dsb, dda, ddb, b1.reshape(2, 1, 128), W2)
    agg2 = _prop_col_kernel(h2, srcoff, dst3)
    w3p = jnp.pad(W3, ((0, 0), (0, 128 - D_OUT)))
    h3 = _mm3(agg2, dsa, dsb, dda, ddb, b2.reshape(2, 1, 128), w3p)
    aggp3 = _prop_edge_kernel(h3, src4, dst4)
    out = _epi(aggp3, dda, ddb, b3.reshape(1, D_OUT))
    return out[:N]
